# baseline (speedup 1.0000x reference)
import functools

import jax
import jax.numpy as jnp
from jax import lax
from jax.experimental import pallas as pl
from jax.experimental.pallas import tpu as pltpu
from jax.experimental.pallas import tpu_sc as plsc

VOCAB = 1000000
D = 64
B = 16384 * 50

_info = plsc.get_sparse_core_info()
NC, NS = _info.num_cores, _info.num_subcores
NW = NC * NS
B_PER_W = B // NW
CHUNK = 320
N_CHUNKS = B_PER_W // CHUNK
NBUF = 4
N_BLOCKS = N_CHUNKS // NBUF


@functools.partial(
    pl.kernel,
    mesh=plsc.VectorSubcoreMesh(core_axis_name="c", subcore_axis_name="s"),
    out_type=jax.ShapeDtypeStruct((B, D), jnp.float32),
    scratch_types=[
        pltpu.VMEM((B_PER_W,), jnp.int32),
        [pltpu.VMEM((CHUNK, D), jnp.float32) for _ in range(NBUF)],
        [pltpu.SemaphoreType.DMA for _ in range(NBUF)],
        [pltpu.SemaphoreType.DMA for _ in range(NBUF)],
    ],
    compiler_params=pltpu.CompilerParams(use_tc_tiling_on_sc=False),
)
def _gather_kernel(table_hbm, idx_hbm, out_hbm, idx_all, rows, sg, so):
    wid = lax.axis_index("s") * NC + lax.axis_index("c")
    base = wid * B_PER_W
    pltpu.sync_copy(idx_hbm.at[pl.ds(base, B_PER_W)], idx_all)

    def remap_body(g, carry):
        v = idx_all[pl.ds(g * 16, 16)]
        p = (v & (-128)) + ((v & 63) << 1) + ((v >> 6) & 1)
        idx_all[pl.ds(g * 16, 16)] = p
        return carry

    lax.fori_loop(0, B_PER_W // 16, remap_body, 0)

    def fire_gather(c, b):
        pltpu.async_copy(
            table_hbm.at[idx_all.at[pl.ds(c * CHUNK, CHUNK)]], rows[b], sg[b]
        )

    def wait_gather(b):
        pltpu.make_async_copy(
            out_hbm.at[pl.ds(base, CHUNK)], rows[b], sg[b]
        ).wait()

    def fire_write(c, b):
        pltpu.async_copy(rows[b], out_hbm.at[pl.ds(base + c * CHUNK, CHUNK)], so[b])

    def wait_write(b):
        pltpu.make_async_copy(
            rows[b], out_hbm.at[pl.ds(base, CHUNK)], so[b]
        ).wait()

    fire_gather(0, 0)
    fire_gather(1, 1)

    wait_gather(0)
    fire_write(0, 0)
    fire_gather(2, 2)
    wait_gather(1)
    fire_write(1, 1)
    fire_gather(3, 3)
    wait_gather(2)
    fire_write(2, 2)
    wait_write(0)
    fire_gather(4, 0)
    wait_gather(3)
    fire_write(3, 3)
    wait_write(1)
    fire_gather(5, 1)

    def body(i, carry):
        c0 = i * NBUF
        for b in range(NBUF):
            wait_gather(b)
            fire_write(c0 + b, b)
            wait_write((b + 2) % NBUF)
            fire_gather(c0 + b + 2, (b + 2) % NBUF)
        return carry

    lax.fori_loop(1, N_BLOCKS - 1, body, 0)

    cl = (N_BLOCKS - 1) * NBUF
    wait_gather(0)
    fire_write(cl, 0)
    wait_write(2)
    fire_gather(cl + 2, 2)
    wait_gather(1)
    fire_write(cl + 1, 1)
    wait_write(3)
    fire_gather(cl + 3, 3)
    wait_gather(2)
    fire_write(cl + 2, 2)
    wait_gather(3)
    fire_write(cl + 3, 3)

    for b in range(NBUF):
        wait_write(b)


BATCH = 16384
HIST = 50
TBLK = 128 * HIST * D // 128


@functools.partial(
    pl.pallas_call,
    grid=(BATCH // 128,),
    in_specs=[pl.BlockSpec((TBLK, 128), lambda i: (i, 0))],
    out_specs=pl.BlockSpec((HIST, 8, 1, 8, 128), lambda i: (0, 0, i, 0, 0)),
    out_shape=jax.ShapeDtypeStruct((HIST, 8, BATCH // 128, 8, 128), jnp.float32),
)
def _retile_kernel(in_ref, out_ref):
    x3 = in_ref[...].reshape(128, HIST // 2, 128)
    for hh in range(HIST // 2):
        yt = x3[:, hh, :].T
        out_ref[2 * hh, :, 0, :, :] = yt[0:64].reshape(8, 8, 128)
        out_ref[2 * hh + 1, :, 0, :, :] = yt[64:128].reshape(8, 8, 128)


VBLK = 1024
NVB = (VOCAB + VBLK - 1) // VBLK
VPAD = NVB * VBLK


@functools.partial(
    pl.pallas_call,
    grid=(NVB,),
    in_specs=[pl.BlockSpec((D, VBLK), lambda i: (0, i))],
    out_specs=pl.BlockSpec((VBLK // 2, 128), lambda i: (i, 0)),
    out_shape=jax.ShapeDtypeStruct((VPAD // 2, 128), jnp.float32),
)
def _detile_kernel(in_ref, out_ref):
    x = in_ref[...]
    for k in range(VBLK // 128):
        y = x[:, k * 128 : (k + 1) * 128].T
        out_ref[k * 64 : k * 64 + 64, 0:D] = y[0:64]
        out_ref[k * 64 : k * 64 + 64, D:128] = y[64:128]


def kernel(x, table):
    idx = x.reshape(-1).astype(jnp.int32)
    table_flat = _detile_kernel(table.T).reshape(VPAD, D)
    out = _gather_kernel(table_flat, idx)
    o5 = _retile_kernel(out.reshape(B * D // 128, 128))
    return o5.transpose(2, 4, 0, 1, 3).reshape(BATCH, HIST, D)


# speedup vs baseline: 1.5032x; 1.5032x over previous
import functools

import jax
import jax.numpy as jnp
from jax import lax
from jax.experimental import pallas as pl
from jax.experimental.pallas import tpu as pltpu
from jax.experimental.pallas import tpu_sc as plsc

VOCAB = 1000000
D = 64
B = 16384 * 50

_info = plsc.get_sparse_core_info()
NC, NS = _info.num_cores, _info.num_subcores
NW = NC * NS
B_PER_W = B // NW
CHUNK = 320
N_CHUNKS = B_PER_W // CHUNK
NBUF = 4
N_BLOCKS = N_CHUNKS // NBUF


@functools.partial(
    pl.kernel,
    mesh=plsc.VectorSubcoreMesh(core_axis_name="c", subcore_axis_name="s"),
    out_type=jax.ShapeDtypeStruct((B, D), jnp.float32),
    scratch_types=[
        pltpu.VMEM((B_PER_W,), jnp.int32),
        [pltpu.VMEM((CHUNK, D), jnp.float32) for _ in range(NBUF)],
        [pltpu.SemaphoreType.DMA for _ in range(NBUF)],
        [pltpu.SemaphoreType.DMA for _ in range(NBUF)],
    ],
    compiler_params=pltpu.CompilerParams(use_tc_tiling_on_sc=False),
)
def _gather_kernel(table_hbm, idx_hbm, out_hbm, idx_all, rows, sg, so):
    wid = lax.axis_index("s") * NC + lax.axis_index("c")
    base = wid * B_PER_W
    pltpu.sync_copy(idx_hbm.at[pl.ds(base, B_PER_W)], idx_all)

    def remap_body(g, carry):
        v = idx_all[pl.ds(g * 16, 16)]
        p = (v & (-128)) + ((v & 63) << 1) + ((v >> 6) & 1)
        idx_all[pl.ds(g * 16, 16)] = p
        return carry

    lax.fori_loop(0, B_PER_W // 16, remap_body, 0)

    def fire_gather(c, b):
        pltpu.async_copy(
            table_hbm.at[idx_all.at[pl.ds(c * CHUNK, CHUNK)]], rows[b], sg[b]
        )

    def wait_gather(b):
        pltpu.make_async_copy(
            out_hbm.at[pl.ds(base, CHUNK)], rows[b], sg[b]
        ).wait()

    def fire_write(c, b):
        pltpu.async_copy(rows[b], out_hbm.at[pl.ds(base + c * CHUNK, CHUNK)], so[b])

    def wait_write(b):
        pltpu.make_async_copy(
            rows[b], out_hbm.at[pl.ds(base, CHUNK)], so[b]
        ).wait()

    fire_gather(0, 0)
    fire_gather(1, 1)

    wait_gather(0)
    fire_write(0, 0)
    fire_gather(2, 2)
    wait_gather(1)
    fire_write(1, 1)
    fire_gather(3, 3)
    wait_gather(2)
    fire_write(2, 2)
    wait_write(0)
    fire_gather(4, 0)
    wait_gather(3)
    fire_write(3, 3)
    wait_write(1)
    fire_gather(5, 1)

    def body(i, carry):
        c0 = i * NBUF
        for b in range(NBUF):
            wait_gather(b)
            fire_write(c0 + b, b)
            wait_write((b + 2) % NBUF)
            fire_gather(c0 + b + 2, (b + 2) % NBUF)
        return carry

    lax.fori_loop(1, N_BLOCKS - 1, body, 0)

    cl = (N_BLOCKS - 1) * NBUF
    wait_gather(0)
    fire_write(cl, 0)
    wait_write(2)
    fire_gather(cl + 2, 2)
    wait_gather(1)
    fire_write(cl + 1, 1)
    wait_write(3)
    fire_gather(cl + 3, 3)
    wait_gather(2)
    fire_write(cl + 2, 2)
    wait_gather(3)
    fire_write(cl + 3, 3)

    for b in range(NBUF):
        wait_write(b)


BATCH = 16384
HIST = 50
TBLK = 128 * HIST * D // 128


@functools.partial(
    pl.pallas_call,
    grid=(BATCH // 128,),
    in_specs=[pl.BlockSpec((TBLK, 128), lambda i: (i, 0))],
    out_specs=pl.BlockSpec((HIST, 8, 1, 8, 128), lambda i: (0, 0, i, 0, 0)),
    out_shape=jax.ShapeDtypeStruct((HIST, 8, BATCH // 128, 8, 128), jnp.float32),
)
def _retile_kernel(in_ref, out_ref):
    x3 = in_ref[...].reshape(128, HIST // 2, 128)
    for hh in range(HIST // 2):
        yt = x3[:, hh, :].T
        out_ref[2 * hh, :, 0, :, :] = yt[0:64].reshape(8, 8, 128)
        out_ref[2 * hh + 1, :, 0, :, :] = yt[64:128].reshape(8, 8, 128)


VBLK = 4096
NVB = (VOCAB + VBLK - 1) // VBLK
VPAD = NVB * VBLK


@functools.partial(
    pl.pallas_call,
    grid=(NVB,),
    in_specs=[pl.BlockSpec((D, VBLK), lambda i: (0, i))],
    out_specs=pl.BlockSpec((VBLK // 2, 128), lambda i: (i, 0)),
    out_shape=jax.ShapeDtypeStruct((VPAD // 2, 128), jnp.float32),
)
def _detile_kernel(in_ref, out_ref):
    x = in_ref[...]
    for k in range(VBLK // 128):
        y = x[:, k * 128 : (k + 1) * 128].T
        out_ref[k * 64 : k * 64 + 64, 0:D] = y[0:64]
        out_ref[k * 64 : k * 64 + 64, D:128] = y[64:128]


def kernel(x, table):
    idx = x.reshape(-1).astype(jnp.int32)
    table_flat = _detile_kernel(table.T).reshape(VPAD, D)
    out = _gather_kernel(table_flat, idx)
    o5 = _retile_kernel(out.reshape(B * D // 128, 128))
    return o5.transpose(2, 4, 0, 1, 3).reshape(BATCH, HIST, D)
